# baseline (speedup 1.0000x reference)
import jax
import jax.numpy as jnp
from jax import lax
from jax.experimental import pallas as pl
from jax.experimental.pallas import tpu as pltpu
from jax.experimental.pallas import tpu_sc as plsc

B, F, L, D = 1024, 26, 50, 32
V = 1000000
NW = 32
BT = B // 128
LSTEP = 5
NLQ = L // LSTEP
STAGES = F * BT * NLQ
SPW = STAGES // NW
NR = 128 * LSTEP
RP = 40
SV = 1000
NSLAB = V // SV
SLAB_ITERS = (NSLAB + NW - 1) // NW


def _tr_body(tt_hbm, tl_hbm, slab_v, outb_v, rs0, rs1):
    wid = lax.axis_index("s") * 2 + lax.axis_index("c")
    rsems = (rs0, rs1)
    iota = lax.iota(jnp.int32, 16)

    def slab_id(k):
        return k * NW + wid

    def issue_read(k, slot):
        @pl.when(slab_id(k) < NSLAB)
        def _():
            pltpu.async_copy(
                tt_hbm.at[:, pl.ds(slab_id(k) * SV, SV)],
                slab_v.at[slot],
                rsems[slot],
            )

    def wait_read(slot):
        pltpu.make_async_copy(
            tt_hbm.at[:, pl.ds(0, SV)], slab_v.at[slot], rsems[slot]
        ).wait()

    issue_read(0, 0)

    @pl.loop(0, SLAB_ITERS + 1, step=2)
    def _(t):
        for b in range(2):
            k = t + b

            @pl.when((k + 1 < SLAB_ITERS) & (slab_id(k + 1) < NSLAB))
            def _():
                issue_read(k + 1, 1 - b)

            @pl.when((k < SLAB_ITERS) & (slab_id(k) < NSLAB))
            def _():
                wait_read(b)
                slab = slab_v.at[b]

                @plsc.parallel_loop(0, SV, unroll=2)
                def _(v):
                    cv = iota * 0 + v
                    for d0 in (0, 16):
                        outb_v[v, pl.ds(d0, 16)] = plsc.load_gather(
                            slab, [iota + d0, cv]
                        )

                pltpu.sync_copy(outb_v, tl_hbm.at[pl.ds(slab_id(k) * SV, SV)])


def _sc_body(idx_hbm, table_hbm, out_hbm, idxr_v, idxt_v, rows_v,
             out_v, gs0, gs1, os0, os1, isem):
    wid = lax.axis_index("s") * 2 + lax.axis_index("c")
    s0 = wid * SPW
    gsems = (gs0, gs1)
    osems = (os0, os1)
    iota = lax.iota(jnp.int32, 16)

    def decode(st):
        lq = st % NLQ
        fb = st // NLQ
        return fb // BT, fb % BT, lq

    def issue_idx(st):
        f, bh, lq = decode(st)
        pltpu.async_copy(
            idx_hbm.at[pl.ds(bh * 128, 128), f], idxr_v, isem
        )

    def wait_idx():
        pltpu.make_async_copy(
            idx_hbm.at[pl.ds(0, 128), 0], idxr_v, isem
        ).wait()

    def issue_gather(st, slot):
        _, _, lq = decode(st)
        tr = idxt_v.at[slot]
        for c in range(LSTEP):
            cv = iota * 0 + (lq * LSTEP + c)
            for r0 in range(0, 128, 16):
                v = plsc.load_gather(idxr_v, [iota + r0, cv])
                tr[c, pl.ds(r0, 16)] = v
        for c in range(LSTEP):
            pltpu.async_copy(
                table_hbm.at[tr.at[c]],
                rows_v.at[slot].at[pl.ds(c * 128, 128)],
                gsems[slot],
            )

    def wait_gather(slot):
        pltpu.make_async_copy(
            table_hbm.at[pl.ds(0, NR)], rows_v.at[slot], gsems[slot]
        ).wait()

    def out_view(st):
        f, bh, lq = decode(st)
        return out_hbm.at[f, pl.ds(lq * LSTEP, LSTEP), :, bh]

    def shuffle(slot):
        rows = rows_v.at[slot]
        outs = out_v.at[slot]

        @plsc.parallel_loop(0, LSTEP * D, unroll=2)
        def _(i):
            lrel = i // D
            d = i - lrel * D
            cv = iota * 0 + d
            for k in range(8):
                v = plsc.load_gather(rows, [iota + (lrel * 128 + k * 16), cv])
                outs[lrel, d // 8, d % 8, pl.ds(k * 16, 16)] = v

    issue_idx(s0)
    wait_idx()
    issue_gather(s0, 0)

    @pl.loop(0, SPW + 1, step=2)
    def _(t):
        for b in range(2):
            st = t + b

            @pl.when(st + 1 < SPW)
            def _():
                issue_idx(s0 + st + 1)
                wait_idx()
                issue_gather(s0 + st + 1, 1 - b)

            @pl.when(st < SPW)
            def _():
                wait_gather(b)

                @pl.when(st >= 2)
                def _():
                    pltpu.make_async_copy(
                        out_v.at[b], out_view(s0 + st - 2), osems[b]
                    ).wait()

                shuffle(b)
                pltpu.async_copy(out_v.at[b], out_view(s0 + st), osems[b])

    sl0 = (SPW - 2) % 2
    sl1 = (SPW - 1) % 2
    pltpu.make_async_copy(out_v.at[sl0], out_view(s0 + SPW - 2), osems[sl0]).wait()
    pltpu.make_async_copy(out_v.at[sl1], out_view(s0 + SPW - 1), osems[sl1]).wait()


def kernel(inputs, table):
    assert inputs.shape == (B, F, L) and table.shape == (V, D)
    idx = inputs.astype(jnp.int32)
    tt = table.T

    mesh = plsc.VectorSubcoreMesh(core_axis_name="c", subcore_axis_name="s")
    cp = pltpu.CompilerParams(
        needs_layout_passes=False, use_tc_tiling_on_sc=False
    )
    table_l = pl.kernel(
        _tr_body,
        out_type=jax.ShapeDtypeStruct((V, RP), jnp.float32),
        mesh=mesh,
        compiler_params=cp,
        scratch_types=[
            pltpu.VMEM((2, D, SV), jnp.float32),
            pltpu.VMEM((SV, RP), jnp.float32),
            pltpu.SemaphoreType.DMA,
            pltpu.SemaphoreType.DMA,
        ],
    )(tt)
    out6 = pl.kernel(
        _sc_body,
        out_type=jax.ShapeDtypeStruct((F, L, 4, 8, 8, 128), jnp.float32),
        mesh=mesh,
        compiler_params=cp,
        scratch_types=[
            pltpu.VMEM((128, L), jnp.int32),
            pltpu.VMEM((2, LSTEP, 128), jnp.int32),
            pltpu.VMEM((2, NR, RP), jnp.float32),
            pltpu.VMEM((2, LSTEP, 4, 8, 128), jnp.float32),
            pltpu.SemaphoreType.DMA,
            pltpu.SemaphoreType.DMA,
            pltpu.SemaphoreType.DMA,
            pltpu.SemaphoreType.DMA,
            pltpu.SemaphoreType.DMA,
        ],
    )(idx, table_l)
    return out6.transpose(3, 5, 0, 2, 4, 1).reshape(B, F, D, L)


# speedup vs baseline: 4.1871x; 4.1871x over previous
import jax
import jax.numpy as jnp
from jax import lax
from jax.experimental import pallas as pl
from jax.experimental.pallas import tpu as pltpu
from jax.experimental.pallas import tpu_sc as plsc

B, F, L, D = 1024, 26, 50, 32
NW = 32
BT = B // 128
LSTEP = 5
NLQ = L // LSTEP
STAGES = F * BT * NLQ
SPW = STAGES // NW
NR = 128 * LSTEP
RP = D + 1


def _sc_body(idx_hbm, table_hbm, out_hbm, idxr_v, idxt_v, rows_v, rowsp_v,
             out_v, gs0, gs1, os0, os1, isem):
    wid = lax.axis_index("s") * 2 + lax.axis_index("c")
    s0 = wid * SPW
    gsems = (gs0, gs1)
    osems = (os0, os1)
    iota = lax.iota(jnp.int32, 16)
    iota_rp = iota * RP

    def decode(st):
        lq = st % NLQ
        fb = st // NLQ
        return fb // BT, fb % BT, lq

    def issue_idx(st):
        f, bh, lq = decode(st)
        pltpu.async_copy(
            idx_hbm.at[pl.ds(bh * 128, 128), f],
            idxr_v,
            isem,
        )

    def wait_idx():
        pltpu.make_async_copy(
            idx_hbm.at[pl.ds(0, 128), 0],
            idxr_v,
            isem,
        ).wait()

    def issue_gather(st, slot):
        _, _, lq = decode(st)
        raw = idxr_v
        tr = idxt_v.at[slot]
        for c in range(LSTEP):
            cv = iota * 0 + (lq * LSTEP + c)
            for r0 in range(0, 128, 16):
                v = plsc.load_gather(raw, [iota + r0, cv])
                tr[c, pl.ds(r0, 16)] = v
        for c in range(LSTEP):
            pltpu.async_copy(
                table_hbm.at[tr.at[c]],
                rows_v.at[slot].at[pl.ds(c * 128, 128)],
                gsems[slot],
            )

    def wait_gather(slot):
        pltpu.make_async_copy(
            table_hbm.at[pl.ds(0, NR)],
            rows_v.at[slot],
            gsems[slot],
        ).wait()

    def out_view(st):
        f, bh, lq = decode(st)
        return out_hbm.at[f, pl.ds(lq * LSTEP, LSTEP), :, bh]

    def shuffle(slot):
        rows = rows_v.at[slot]
        outs = out_v.at[slot]

        @plsc.parallel_loop(0, NR, unroll=2)
        def _(r):
            base = r * RP
            for d0 in (0, 16):
                sv = iota + (base + d0)
                plsc.store_scatter(rowsp_v, [sv], rows[r, pl.ds(d0, 16)])

        @plsc.parallel_loop(0, LSTEP * D, unroll=2)
        def _(i):
            lrel = i // D
            d = i - lrel * D
            for k in range(8):
                gv = iota_rp + ((lrel * 128 + k * 16) * RP + d)
                v = plsc.load_gather(rowsp_v, [gv])
                outs[lrel, d // 8, d % 8, pl.ds(k * 16, 16)] = v

    issue_idx(s0)
    wait_idx()
    issue_gather(s0, 0)

    @pl.loop(0, SPW + 1, step=2)
    def _(t):
        for b in range(2):
            st = t + b

            @pl.when(st + 1 < SPW)
            def _():
                @pl.when((s0 + st + 1) % NLQ == 0)
                def _():
                    issue_idx(s0 + st + 1)
                    wait_idx()

                issue_gather(s0 + st + 1, 1 - b)

            @pl.when(st < SPW)
            def _():
                wait_gather(b)

                @pl.when(st >= 2)
                def _():
                    pltpu.make_async_copy(
                        out_v.at[b], out_view(s0 + st - 2), osems[b]
                    ).wait()

                shuffle(b)
                pltpu.async_copy(out_v.at[b], out_view(s0 + st), osems[b])

    sl0 = (SPW - 2) % 2
    sl1 = (SPW - 1) % 2
    pltpu.make_async_copy(out_v.at[sl0], out_view(s0 + SPW - 2), osems[sl0]).wait()
    pltpu.make_async_copy(out_v.at[sl1], out_view(s0 + SPW - 1), osems[sl1]).wait()


def kernel(inputs, table):
    assert inputs.shape == (B, F, L) and table.shape[1] == D
    idx = inputs.astype(jnp.int32)

    mesh = plsc.VectorSubcoreMesh(core_axis_name="c", subcore_axis_name="s")
    out6 = pl.kernel(
        _sc_body,
        out_type=jax.ShapeDtypeStruct((F, L, 4, 8, 8, 128), jnp.float32),
        mesh=mesh,
        compiler_params=pltpu.CompilerParams(
            needs_layout_passes=False, use_tc_tiling_on_sc=False
        ),
        scratch_types=[
            pltpu.VMEM((128, L), jnp.int32),
            pltpu.VMEM((2, LSTEP, 128), jnp.int32),
            pltpu.VMEM((2, NR, D), jnp.float32),
            pltpu.VMEM((NR * RP,), jnp.float32),
            pltpu.VMEM((2, LSTEP, 4, 8, 128), jnp.float32),
            pltpu.SemaphoreType.DMA,
            pltpu.SemaphoreType.DMA,
            pltpu.SemaphoreType.DMA,
            pltpu.SemaphoreType.DMA,
            pltpu.SemaphoreType.DMA,
        ],
    )(idx, table)
    return out6.transpose(3, 5, 0, 2, 4, 1).reshape(B, F, D, L)
